# baseline (speedup 1.0000x reference)
import functools

import jax
import jax.numpy as jnp
from jax import lax
from jax.experimental import pallas as pl
from jax.experimental.pallas import tpu as pltpu
from jax.experimental.pallas import tpu_sc as plsc

B = 16384
F = 64
L = 16
NC, NS = 2, 16
NW = NC * NS
BPW = B // NW
CHUNK = 128
N_CHUNKS = BPW // CHUNK
N_BLOCKS = BPW // L


def _sc_body(ui_hbm, pu_hbm, qi_hbm, pred_hbm, feat_hbm,
             ui_v, uid_v, iid_v, pu_v, qi_v, pred_v, sem):
    wid = lax.axis_index("s") * NC + lax.axis_index("c")
    base = wid * BPW

    pltpu.sync_copy(ui_hbm.at[pl.ds(base, BPW)], ui_v)

    lanes = lax.iota(jnp.int32, L)
    zeros = jnp.zeros((L,), jnp.int32)
    ones = jnp.ones((L,), jnp.int32)
    for blk in range(N_BLOCKS):
        rows = lanes + (blk * L)
        uid_v[pl.ds(blk * L, L)] = plsc.load_gather(ui_v, [rows, zeros])
        iid_v[pl.ds(blk * L, L)] = plsc.load_gather(ui_v, [rows, ones])

    copies = []
    for j in range(N_CHUNKS):
        sl = pl.ds(j * CHUNK, CHUNK)
        copies.append(pltpu.async_copy(
            pu_hbm.at[uid_v.at[sl]], pu_v.at[sl], sem))
        copies.append(pltpu.async_copy(
            qi_hbm.at[iid_v.at[sl]], qi_v.at[sl], sem))
    for c in copies:
        c.wait()

    def blk_body(blk, _):
        acc16 = jnp.zeros((L,), jnp.float32)
        for r16 in range(L):
            r = blk * L + r16
            acc = pu_v[r, pl.ds(0, L)] * qi_v[r, pl.ds(0, L)]
            for c in range(1, F // L):
                acc = acc + pu_v[r, pl.ds(c * L, L)] * qi_v[r, pl.ds(c * L, L)]
            s = jnp.sum(acc)
            acc16 = jnp.where(lanes == r16, s, acc16)
        acc16 = jnp.minimum(jnp.maximum(acc16, 1.0), 5.0)
        pred_v[pl.ds(blk * L, L)] = acc16
        return 0

    lax.fori_loop(0, N_BLOCKS, blk_body, 0)

    pltpu.sync_copy(pred_v, pred_hbm.at[pl.ds(base, BPW)])
    pltpu.sync_copy(pu_v, feat_hbm.at[pl.ds(base, BPW), pl.ds(0, F)])
    pltpu.sync_copy(qi_v, feat_hbm.at[pl.ds(base, BPW), pl.ds(F, F)])


@jax.jit
def _run(user_item, pu, qi):
    mesh = plsc.VectorSubcoreMesh(core_axis_name="c", subcore_axis_name="s")
    return pl.kernel(
        _sc_body,
        out_type=(
            jax.ShapeDtypeStruct((B,), jnp.float32),
            jax.ShapeDtypeStruct((B, 2 * F), jnp.float32),
        ),
        mesh=mesh,
        compiler_params=pltpu.CompilerParams(use_tc_tiling_on_sc=False,
                                             needs_layout_passes=False),
        scratch_types=[
            pltpu.VMEM((BPW, 2), jnp.int32),
            pltpu.VMEM((BPW,), jnp.int32),
            pltpu.VMEM((BPW,), jnp.int32),
            pltpu.VMEM((BPW, F), jnp.float32),
            pltpu.VMEM((BPW, F), jnp.float32),
            pltpu.VMEM((BPW,), jnp.float32),
            pltpu.SemaphoreType.DMA,
        ],
    )(user_item, pu, qi)


def kernel(user_item, pu, qi):
    return _run(user_item.astype(jnp.int32), pu, qi)
